# baseline (speedup 1.0000x reference)
import functools

import jax
import jax.numpy as jnp
from jax import lax
from jax.experimental import pallas as pl
from jax.experimental.pallas import tpu as pltpu
from jax.experimental.pallas import tpu_sc as plsc

VOCAB = 33
HIDDEN = 512
NUM_TABLES = 6
B = 4 * 8192
NC, NS = 2, 16
NW = NC * NS
L = 16
K = HIDDEN // L
N_SC = 3
N_TC = NUM_TABLES - N_SC
ROWS_PER_W = B // NW
CHUNK = 32
NCHUNK = ROWS_PER_W // CHUNK
NBUF = 3
TC_BLK = 512


def _make_sc_lookup():
  mesh = plsc.VectorSubcoreMesh(
      core_axis_name="c", subcore_axis_name="s", num_cores=NC, num_subcores=NS
  )
  out_type = [
      jax.ShapeDtypeStruct((B, HIDDEN), jnp.float32) for _ in range(N_SC)
  ]
  scratch = [
      pltpu.VMEM((ROWS_PER_W // 128, 128), jnp.int32),
      [pltpu.VMEM((VOCAB * HIDDEN,), jnp.float32) for _ in range(N_SC)],
      [pltpu.VMEM((CHUNK, HIDDEN), jnp.float32) for _ in range(NBUF)],
      pltpu.SemaphoreType.DMA,
  ]

  @functools.partial(
      pl.kernel, mesh=mesh, out_type=out_type, scratch_types=scratch,
      compiler_params=pltpu.CompilerParams(needs_layout_passes=False),
  )
  def lookup(idx_hbm, t0, t1, t2, o0, o1, o2, idx_v, tbls, bufs, ssem):
    outs = (o0, o1, o2)
    wid = lax.axis_index("s") * NC + lax.axis_index("c")
    base0 = wid * ROWS_PER_W

    pltpu.sync_copy(idx_hbm.at[wid], idx_v)
    for src, dst in zip((t0, t1, t2), tbls):
      pltpu.sync_copy(src, dst)

    cols = [jnp.arange(L, dtype=jnp.int32) + L * k for k in range(K)]

    def wait_store():
      pltpu.make_async_copy(bufs[0], o0.at[pl.ds(0, CHUNK)], ssem).wait()

    def chunk_body(ci, carry):
      pos0 = ci * CHUNK
      for t in range(N_SC):
        @pl.when(ci > 0)
        def _():
          wait_store()

        buf = bufs[t]

        @plsc.parallel_loop(0, CHUNK, 1, unroll=2)
        def _(j, t=t, buf=buf, pos0=pos0):
          pos = pos0 + j
          bc = plsc.load_gather(
              idx_v,
              [jnp.full((L,), pos // 128, jnp.int32),
               jnp.full((L,), pos % 128, jnp.int32)],
          )
          flat0 = bc * HIDDEN
          for k in range(K):
            buf[j, pl.ds(L * k, L)] = plsc.load_gather(
                tbls[t], [flat0 + cols[k]])

        pltpu.async_copy(buf, outs[t].at[pl.ds(base0 + pos0, CHUNK)], ssem)
      return carry

    lax.fori_loop(0, NCHUNK, chunk_body, 0)
    for _ in range(NBUF):
      wait_store()

  return lookup


def _tc_body(idx_ref, tbl_ref, o0, o1, o2):
  idxv = idx_ref[0, 0, :]
  oh = (idxv[:, None] == lax.broadcasted_iota(jnp.int32, (TC_BLK, VOCAB), 1))
  oh = oh.astype(jnp.float32)
  for t, out in enumerate((o0, o1, o2)):
    out[...] = lax.dot_general(
        oh, tbl_ref[t], (((1,), (0,)), ((), ())),
        preferred_element_type=jnp.float32,
    )


def _tc_onehot(idx3d, tbl_stack):
  grid = (B // TC_BLK,)
  return pl.pallas_call(
      _tc_body,
      grid=grid,
      in_specs=[
          pl.BlockSpec((1, 1, TC_BLK), lambda b: (b, 0, 0)),
          pl.BlockSpec((N_TC, VOCAB, HIDDEN), lambda b: (0, 0, 0)),
      ],
      out_specs=[
          pl.BlockSpec((TC_BLK, HIDDEN), lambda b: (b, 0))
          for _ in range(N_TC)
      ],
      out_shape=[
          jax.ShapeDtypeStruct((B, HIDDEN), jnp.float32) for _ in range(N_TC)
      ],
  )(idx3d, tbl_stack)


_sc_lookup = _make_sc_lookup()


def kernel(inputs, tables):
  idx_sc = inputs.reshape(NW, ROWS_PER_W // 128, 128).astype(jnp.int32)
  idx_tc = inputs.reshape(B // TC_BLK, 1, TC_BLK).astype(jnp.int32)
  sc_out = _sc_lookup(idx_sc, *(tables[N_TC + i].reshape(-1)
                                for i in range(N_SC)))
  tc_out = _tc_onehot(idx_tc, tables[:N_TC])
  flat = list(tc_out) + list(sc_out)
  ve = [o.reshape(inputs.shape + (HIDDEN,)) for o in flat]
  return tuple(ve + list(reversed(ve)))
